# Initial kernel scaffold; baseline (speedup 1.0000x reference)
import jax
import jax.numpy as jnp
from jax.experimental import pallas as pl


def kernel(x, adj_indices, adj_values, ws, idx):
    raise NotImplementedError("write your pallas kernel here")


# speedup vs baseline: 3.9202x; 3.9202x over previous
import functools

import jax
import jax.numpy as jnp
from jax import lax
from jax.experimental import pallas as pl
from jax.experimental.pallas import tpu as pltpu
from jax.experimental.pallas import tpu_sc as plsc

N, D = 10000, 128
NC, NS = 2, 16
NW = NC * NS
L = 16
C = 80
RB = 624
ZR = 16


def _sc_spmm(row, col, vals, x, wsb):
    E = row.shape[0]
    epw = E // NW
    nch = epw // C

    mesh = plsc.VectorSubcoreMesh(core_axis_name="c", subcore_axis_name="s")

    @functools.partial(
        pl.kernel,
        out_type=jax.ShapeDtypeStruct((NC, N, D), jnp.float32),
        mesh=mesh,
        scratch_types=[
            pltpu.VMEM((C,), jnp.int32),
            pltpu.VMEM((C,), jnp.int32),
            pltpu.VMEM((C,), jnp.float32),
            pltpu.VMEM((C, D), jnp.float32),
            pltpu.VMEM((ZR, D), jnp.float32),
            pltpu.VMEM((L,), jnp.float32),
            pltpu.VMEM_SHARED((N, D), jnp.float32),
            pltpu.SemaphoreType.DMA,
        ],
    )
    def k(row_h, col_h, vals_h, x_h, wsb_h, out_h,
          colv, rowv, valv, rows, zbuf, wsv, acc, sem):
        cid = lax.axis_index("c")
        sid = lax.axis_index("s")
        wid = sid * NC + cid
        base = wid * epw

        for i in range(ZR):
            for j in range(D // L):
                zbuf[i, pl.ds(j * L, L)] = jnp.zeros((L,), jnp.float32)
        nblk = jnp.where(sid == NS - 1, (N - (NS - 1) * RB) // ZR, RB // ZR)

        def zblk(t, _):
            pltpu.sync_copy(zbuf, acc.at[pl.ds(sid * RB + t * ZR, ZR)])
            return 0
        lax.fori_loop(0, nblk, zblk, 0)
        plsc.subcore_barrier()

        pltpu.sync_copy(wsb_h, wsv)
        ws_vec = wsv[...]

        def chunk(kk, _):
            off = base + kk * C
            pltpu.sync_copy(row_h.at[pl.ds(off, C)], rowv)
            pltpu.sync_copy(col_h.at[pl.ds(off, C)], colv)
            pltpu.sync_copy(vals_h.at[pl.ds(off, C)], valv)
            pltpu.async_copy(x_h.at[colv], rows, sem).wait()
            for g in range(C // L):
                v = valv[pl.ds(g * L, L)] * ws_vec
                for t in range(L):
                    b = v[t]
                    e = g * L + t
                    for j in range(D // L):
                        sl = (e, pl.ds(j * L, L))
                        rows[sl] = rows[sl] * b

            pltpu.sync_copy(rows, acc.at[rowv], add=True)
            return 0
        lax.fori_loop(0, nch, chunk, 0)

        plsc.subcore_barrier()

        def oblk(t, _):
            s = sid * RB + t * ZR
            pltpu.sync_copy(acc.at[pl.ds(s, ZR)], out_h.at[cid, pl.ds(s, ZR)])
            return 0
        lax.fori_loop(0, nblk, oblk, 0)

    return k(row, col, vals, x, wsb)


def _combine_body(p_ref, o_ref):
    o_ref[...] = p_ref[0] + p_ref[1]


def _combine(partials):
    blk = 1000
    return pl.pallas_call(
        _combine_body,
        out_shape=jax.ShapeDtypeStruct((N, D), jnp.float32),
        grid=(N // blk,),
        in_specs=[pl.BlockSpec((NC, blk, D), lambda i: (0, i, 0))],
        out_specs=pl.BlockSpec((blk, D), lambda i: (i, 0)),
    )(partials)


def kernel(x, adj_indices, adj_values, ws, idx):
    row = adj_indices[idx, 0]
    col = adj_indices[idx, 1]
    vals = adj_values[idx]
    wsb = jnp.broadcast_to(ws[idx], (L,))
    partials = _sc_spmm(row, col, vals, x, wsb)
    return _combine(partials)


# speedup vs baseline: 8.0002x; 2.0408x over previous
import functools

import jax
import jax.numpy as jnp
from jax import lax
from jax.experimental import pallas as pl
from jax.experimental.pallas import tpu as pltpu
from jax.experimental.pallas import tpu_sc as plsc

N, D = 10000, 128
NC, NS = 2, 16
NW = NC * NS
L = 16
C = 80
NBUF = 4
RB = 624
ZR = 16


def _sc_spmm(row, col, vals, x, wsb):
    E = row.shape[0]
    epw = E // NW
    nch = epw // C
    nout = nch // NBUF

    mesh = plsc.VectorSubcoreMesh(core_axis_name="c", subcore_axis_name="s")

    @functools.partial(
        pl.kernel,
        out_type=jax.ShapeDtypeStruct((NC, N, D), jnp.float32),
        mesh=mesh,
        scratch_types=[
            [pltpu.VMEM((C,), jnp.int32) for _ in range(NBUF)],
            [pltpu.VMEM((C,), jnp.int32) for _ in range(NBUF)],
            [pltpu.VMEM((C,), jnp.float32) for _ in range(NBUF)],
            [pltpu.VMEM((C, D), jnp.float32) for _ in range(NBUF)],
            pltpu.VMEM((ZR, D), jnp.float32),
            pltpu.VMEM((L,), jnp.float32),
            pltpu.VMEM_SHARED((N, D), jnp.float32),
            pltpu.SemaphoreType.DMA((NBUF,)),
            pltpu.SemaphoreType.DMA((NBUF,)),
            pltpu.SemaphoreType.DMA((NBUF,)),
        ],
    )
    def k(row_h, col_h, vals_h, x_h, wsb_h, out_h,
          colm, rowm, valm, rows, zbuf, wsv, acc, msem, gsem, ssem):
        cid = lax.axis_index("c")
        sid = lax.axis_index("s")
        wid = sid * NC + cid
        base = wid * epw

        pltpu.sync_copy(wsb_h, wsv)
        ws_vec = wsv[...]

        for i in range(ZR):
            for j in range(D // L):
                zbuf[i, pl.ds(j * L, L)] = jnp.zeros((L,), jnp.float32)
        nblk = jnp.where(sid == NS - 1, (N - (NS - 1) * RB) // ZR, RB // ZR)

        def zblk(t, _):
            pltpu.sync_copy(zbuf, acc.at[pl.ds(sid * RB + t * ZR, ZR)])
            return 0
        lax.fori_loop(0, nblk, zblk, 0)
        plsc.subcore_barrier()

        def issue_meta(b, kk):
            off = base + kk * C
            pltpu.async_copy(row_h.at[pl.ds(off, C)], rowm[b], msem.at[b])
            pltpu.async_copy(col_h.at[pl.ds(off, C)], colm[b], msem.at[b])
            pltpu.async_copy(vals_h.at[pl.ds(off, C)], valm[b], msem.at[b])

        def wait_meta(b, kk):
            off = base + kk * C
            pltpu.make_async_copy(row_h.at[pl.ds(off, C)], rowm[b],
                                  msem.at[b]).wait()
            pltpu.make_async_copy(col_h.at[pl.ds(off, C)], colm[b],
                                  msem.at[b]).wait()
            pltpu.make_async_copy(vals_h.at[pl.ds(off, C)], valm[b],
                                  msem.at[b]).wait()

        def issue_gather(b):
            pltpu.async_copy(x_h.at[colm[b]], rows[b], gsem.at[b])

        def wait_gather(b):
            pltpu.make_async_copy(x_h.at[colm[b]], rows[b], gsem.at[b]).wait()

        def issue_scatter(b):
            pltpu.async_copy(rows[b], acc.at[rowm[b]], ssem.at[b], add=True)

        def wait_scatter(b):
            pltpu.make_async_copy(rows[b], acc.at[rowm[b]], ssem.at[b]).wait()

        def scale(b):
            def grp(g, _):
                v = valm[b][pl.ds(g * L, L)] * ws_vec
                for t in range(L):
                    sc = v[t]
                    e = g * L + t
                    for j in range(D // L):
                        sl = (e, pl.ds(j * L, L))
                        rows[b][sl] = rows[b][sl] * sc
                return 0
            lax.fori_loop(0, C // L, grp, 0)

        for b in range(NBUF):
            issue_meta(b, b)
        for b in range(NBUF):
            wait_meta(b, b)
            issue_gather(b)
        for b in range(NBUF):
            wait_gather(b)
            scale(b)
            issue_scatter(b)

        def round_(r, _):
            for b in range(NBUF):
                wait_scatter(b)
                issue_meta(b, r * NBUF + b)
            for b in range(NBUF):
                wait_meta(b, r * NBUF + b)
                issue_gather(b)
            for b in range(NBUF):
                wait_gather(b)
                scale(b)
                issue_scatter(b)
            return 0
        lax.fori_loop(1, nout, round_, 0)

        for kk in range(nout * NBUF, nch):
            wait_scatter(0)
            issue_meta(0, kk)
            wait_meta(0, kk)
            issue_gather(0)
            wait_gather(0)
            scale(0)
            issue_scatter(0)

        for b in range(NBUF):
            wait_scatter(b)

        plsc.subcore_barrier()

        def oblk(t, _):
            s = sid * RB + t * ZR
            pltpu.sync_copy(acc.at[pl.ds(s, ZR)], out_h.at[cid, pl.ds(s, ZR)])
            return 0
        lax.fori_loop(0, nblk, oblk, 0)

    return k(row, col, vals, x, wsb)


def _combine_body(p_ref, o_ref):
    o_ref[...] = p_ref[0] + p_ref[1]


def _combine(partials):
    blk = 1000
    return pl.pallas_call(
        _combine_body,
        out_shape=jax.ShapeDtypeStruct((N, D), jnp.float32),
        grid=(N // blk,),
        in_specs=[pl.BlockSpec((NC, blk, D), lambda i: (0, i, 0))],
        out_specs=pl.BlockSpec((blk, D), lambda i: (i, 0)),
    )(partials)


def kernel(x, adj_indices, adj_values, ws, idx):
    row = adj_indices[idx, 0]
    col = adj_indices[idx, 1]
    vals = adj_values[idx]
    wsb = jnp.broadcast_to(ws[idx], (L,))
    partials = _sc_spmm(row, col, vals, x, wsb)
    return _combine(partials)
